# Initial kernel scaffold; baseline (speedup 1.0000x reference)
import jax
import jax.numpy as jnp
from jax.experimental import pallas as pl


def kernel(x, Wg, W1, W2, W3):
    raise NotImplementedError("write your pallas kernel here")


# speedup vs baseline: 1.4414x; 1.4414x over previous
import jax
import jax.numpy as jnp
from jax import lax
from jax.experimental import pallas as pl
from jax.experimental.pallas import tpu as pltpu

B, S, D = 32, 8, 768
E = 16
H = 2 * D
N = B * S
TB = 32
NBLK = 24

_F32 = jnp.float32
_I32 = jnp.int32


def _dot(a, b, dims):
    return lax.dot_general(a, b, (dims, ((), ())), preferred_element_type=_F32)


def _router_kernel(x_ref, wg_ref, pos_ref, be_ref, base_ref, limit_ref):
    x = x_ref[...]
    wg = wg_ref[...]
    gate = _dot(x, wg, ((1,), (1,)))

    e_iota = lax.broadcasted_iota(_I32, (N, E), 1)
    mx = jnp.max(gate, axis=1, keepdims=True)
    eid = jnp.min(jnp.where(gate == mx, e_iota, E), axis=1, keepdims=True)
    oh = (e_iota == eid).astype(_F32)

    cnt = jnp.sum(oh, axis=0, keepdims=True)
    lt16 = (lax.broadcasted_iota(_I32, (E, E), 0)
            < lax.broadcasted_iota(_I32, (E, E), 1)).astype(_F32)
    off = _dot(cnt, lt16, ((1,), (0,)))
    le256 = (lax.broadcasted_iota(_I32, (N, N), 1)
             <= lax.broadcasted_iota(_I32, (N, N), 0)).astype(_F32)
    ranks = _dot(le256, oh, ((1,), (0,)))
    rank = jnp.sum(ranks * oh, axis=1, keepdims=True)
    off_tok = jnp.sum(off * oh, axis=1, keepdims=True)
    pos = off_tok + rank - 1.0
    pos_ref[...] = pos.astype(_I32)

    cnt_i = cnt.astype(_I32)
    nblk = (cnt_i + (TB - 1)) // TB
    blkstart = _dot(nblk.astype(_F32), lt16, ((1,), (0,)))
    b_iota = lax.broadcasted_iota(_I32, (NBLK, E), 0).astype(_F32)
    e_iota2 = lax.broadcasted_iota(_I32, (NBLK, E), 1)
    cond = (blkstart <= b_iota) & (nblk > 0)
    bev = jnp.max(jnp.where(cond, e_iota2, -1), axis=1, keepdims=True)
    ohb = (e_iota2 == bev).astype(_F32)
    bs_b = jnp.sum(blkstart * ohb, axis=1, keepdims=True)
    off_b = jnp.sum(off * ohb, axis=1, keepdims=True)
    cnt_b = jnp.sum(cnt * ohb, axis=1, keepdims=True)
    lb = lax.broadcasted_iota(_I32, (NBLK, 1), 0).astype(_F32) - bs_b
    base = off_b + lb * TB
    limit = off_b + cnt_b
    be_ref[...] = bev.astype(_I32)
    base_ref[...] = base.astype(_I32)
    limit_ref[...] = limit.astype(_I32)


def _expert_kernel(be_s, base_s, limit_s, x_ref, pos_ref,
                   w1_ref, w2_ref, w3_ref, out_ref):
    b = pl.program_id(0)
    base = base_s[b]
    limit = limit_s[b]
    posv = pos_ref[...]
    r_iota = lax.broadcasted_iota(_I32, (N, TB), 1)
    p2 = ((posv - base == r_iota) & (posv < limit)).astype(_F32)

    xblk = _dot(p2, x_ref[...], ((0,), (0,)))
    w1 = w1_ref[0]
    w2 = w2_ref[0]
    w3 = w3_ref[0]
    h = _dot(xblk, w1, ((1,), (1,)))
    v = _dot(xblk, w2, ((1,), (1,)))
    act = h * jax.nn.sigmoid(h) * v
    y = _dot(act, w3, ((1,), (1,)))

    @pl.when(b == 0)
    def _():
        out_ref[...] = jnp.zeros_like(out_ref)

    out_ref[...] += _dot(p2, y, ((1,), (0,)))


def kernel(x, Wg, W1, W2, W3):
    x2 = x.reshape(N, D)
    pos, be, base, limit = pl.pallas_call(
        _router_kernel,
        out_shape=[
            jax.ShapeDtypeStruct((N, 1), _I32),
            jax.ShapeDtypeStruct((NBLK, 1), _I32),
            jax.ShapeDtypeStruct((NBLK, 1), _I32),
            jax.ShapeDtypeStruct((NBLK, 1), _I32),
        ],
    )(x2, Wg)

    grid_spec = pltpu.PrefetchScalarGridSpec(
        num_scalar_prefetch=3,
        grid=(NBLK,),
        in_specs=[
            pl.BlockSpec((N, D), lambda b, be_r, ba_r, li_r: (0, 0)),
            pl.BlockSpec((N, 1), lambda b, be_r, ba_r, li_r: (0, 0)),
            pl.BlockSpec((1, H, D), lambda b, be_r, ba_r, li_r: (be_r[b], 0, 0)),
            pl.BlockSpec((1, H, D), lambda b, be_r, ba_r, li_r: (be_r[b], 0, 0)),
            pl.BlockSpec((1, D, H), lambda b, be_r, ba_r, li_r: (be_r[b], 0, 0)),
        ],
        out_specs=pl.BlockSpec((N, D), lambda b, be_r, ba_r, li_r: (0, 0)),
    )
    out = pl.pallas_call(
        _expert_kernel,
        grid_spec=grid_spec,
        out_shape=jax.ShapeDtypeStruct((N, D), _F32),
        compiler_params=pltpu.CompilerParams(
            dimension_semantics=("arbitrary",),
        ),
    )(be.reshape(NBLK), base.reshape(NBLK), limit.reshape(NBLK),
      x2, pos, W1, W2, W3)
    return out.reshape(B, S, D)
